# baseline (speedup 1.0000x reference)
import functools

import jax
import jax.numpy as jnp
import numpy as np
from jax import lax
from jax.experimental import pallas as pl
from jax.experimental.pallas import tpu as pltpu
from jax.experimental.pallas import tpu_sc as plsc

NC = 2
NS = 16
L = 16
NW = NC * NS

C = 128
H = 128
W = 128
PB = 64
NG = PB // L
PLANES = 3
OB = PB * PLANES * C

_DEN = 1 + 0.1 + 10e-4
_CLAMP_HI = 1 - 10e-4

OC = PLANES * C // 2


def _coords_to_cells(a):
    u = a / jnp.float32(_DEN)
    u = u + jnp.float32(0.5)
    u = jnp.where(u >= jnp.float32(1.0), jnp.float32(_CLAMP_HI), u)
    u = jnp.where(u < jnp.float32(0.0), jnp.float32(0.0), u)
    vg = jnp.float32(2.0) * u - jnp.float32(1.0)
    x = (vg + jnp.float32(1.0)) * jnp.float32(0.5) * jnp.float32(W - 1)
    x = jnp.minimum(jnp.maximum(x, jnp.float32(0.0)), jnp.float32(W - 1))
    c0 = x.astype(jnp.int32)
    w = x - c0.astype(jnp.float32)
    c0 = jnp.minimum(c0, jnp.int32(W - 2))
    return c0, w


def _bf16x2(v):
    u = lax.bitcast_convert_type(v, jnp.int32)
    lo = lax.bitcast_convert_type(u << jnp.int32(16), jnp.float32)
    hi = lax.bitcast_convert_type(u & jnp.int32(-65536), jnp.float32)
    return lo, hi


def _sc_body(px_hbm, py_hbm, pz_hbm, table_hbm, out_hbm,
             cx, cy, cz, idx2, w2, r2, out2, gsem, osem,
             *, pts_per_w, base_step, base_max):
    wid = lax.axis_index("s") * NC + lax.axis_index("c")
    base = pl.multiple_of(
        jnp.minimum(wid * base_step, base_max) & jnp.int32(-8), 8)
    nblk = pts_per_w // PB
    npair = nblk // 2

    pltpu.sync_copy(px_hbm.at[pl.ds(base, pts_per_w)], cx)
    pltpu.sync_copy(py_hbm.at[pl.ds(base, pts_per_w)], cy)
    pltpu.sync_copy(pz_hbm.at[pl.ds(base, pts_per_w)], cz)

    plane_coords = [(cx, cz), (cx, cy), (cy, cz)]

    def compute_idx(par, plane, loc):
        ca, cb = plane_coords[plane]
        w00, w01, w10, w11 = w2[par]
        for g in range(NG):
            sl = pl.ds(loc + g * L, L)
            so = pl.ds(g * L, L)
            sf = pl.ds(0, L)
            x0, wx = _coords_to_cells(ca[sl])
            y0, wy = _coords_to_cells(cb[sl])
            idx = (y0 * W + x0) + jnp.int32(plane * H * W)
            idx2[par][g][sf] = idx
            idx2[par][NG + g][sf] = idx + W
            one = jnp.float32(1.0)
            w00[so] = (one - wx) * (one - wy)
            w01[so] = wx * (one - wy)
            w10[so] = (one - wx) * wy
            w11[so] = wx * wy

    def fire(par):
        for k in range(2 * NG):
            pltpu.make_async_copy(table_hbm.at[idx2[par][k]], r2[par][k],
                                  gsem[par]).start()

    def drain(par):
        for k in range(2 * NG):
            pltpu.make_async_copy(table_hbm.at[idx2[par][k]], r2[par][k],
                                  gsem[par]).wait()

    def consume(par, plane, out_v):
        w00, w01, w10, w11 = w2[par]
        for g in range(NG):
            sg = pl.ds(g * L, L)
            wv00 = w00[sg]
            wv01 = w01[sg]
            wv10 = w10[sg]
            wv11 = w11[sg]
            r0 = r2[par][g]
            r1 = r2[par][NG + g]

            def pt_body(ii, _, plane=plane, g=g, r0=r0, r1=r1,
                        wv00=wv00, wv01=wv01, wv10=wv10, wv11=wv11):
                i16 = jnp.full((L,), ii, jnp.int32)
                w00s = wv00.at[i16].get(mode="promise_in_bounds")
                w01s = wv01.at[i16].get(mode="promise_in_bounds")
                w10s = wv10.at[i16].get(mode="promise_in_bounds")
                w11s = wv11.at[i16].get(mode="promise_in_bounds")
                i = g * L + ii
                for j in range(C // (2 * L)):
                    cs = pl.ds(j * L, L)
                    cs1 = pl.ds(C // 2 + j * L, L)
                    a00, b00 = _bf16x2(r0[ii, cs])
                    a01, b01 = _bf16x2(r0[ii, cs1])
                    a10, b10 = _bf16x2(r1[ii, cs])
                    a11, b11 = _bf16x2(r1[ii, cs1])
                    acc_a = (w00s * a00 + w01s * a01
                             + w10s * a10 + w11s * a11)
                    acc_b = (w00s * b00 + w01s * b01
                             + w10s * b10 + w11s * b11)
                    ua = lax.bitcast_convert_type(acc_a, jnp.int32)
                    ub = lax.bitcast_convert_type(acc_b, jnp.int32)
                    half = jnp.int32(0x8000)
                    lo16 = ((ua + half) >> jnp.int32(16)) & jnp.int32(0xFFFF)
                    hi16 = (ub + half) & jnp.int32(-65536)
                    pk = lax.bitcast_convert_type(lo16 | hi16, jnp.float32)
                    out_v[i, pl.ds(plane * (C // 2) + j * L, L)] = pk
                return 0

            lax.fori_loop(0, L, pt_body, 0)

    compute_idx(0, 0, 0)
    fire(0)

    def pair_body(i, _):
        for k in range(6):
            plane = k % 3
            boff = k // 3
            par = k % 2
            nxt = (k + 1) % 2
            blk = 2 * i + boff
            out_v = out2[boff]
            nplane = (k + 1) % 3
            nboff = (k + 1) // 3
            if k < 5:
                compute_idx(nxt, nplane, (2 * i + nboff) * PB)
                fire(nxt)
            else:
                @pl.when(i < npair - 1)
                def _():
                    compute_idx(nxt, 0, (2 * i + 2) * PB)
                    fire(nxt)
            drain(par)
            if plane == 0:
                @pl.when(i > 0)
                def _():
                    pltpu.make_async_copy(
                        out_v, out_hbm.at[pl.ds(base, PB)], osem[boff]).wait()
            consume(par, plane, out_v)
            if plane == 2:
                pltpu.make_async_copy(
                    out_v, out_hbm.at[pl.ds(base + blk * PB, PB)],
                    osem[boff]).start()
        return 0

    lax.fori_loop(0, npair, pair_body, 0)

    for boff in range(2):
        pltpu.make_async_copy(out2[boff], out_hbm.at[pl.ds(base, PB)],
                              osem[boff]).wait()


@functools.partial(jax.jit, static_argnames=("n", "pts_per_w"))
def _sc_sample(px, py, pz, table, n, pts_per_w):
    base_step = -(-(n - pts_per_w) // (NW - 1))
    base_max = n - pts_per_w
    mesh = plsc.VectorSubcoreMesh(
        core_axis_name="c", subcore_axis_name="s", num_cores=NC,
        num_subcores=NS)

    def full_body(px_hbm, py_hbm, pz_hbm, table_hbm, out_hbm, *scratch):
        cx, cy, cz = scratch[0:3]
        o = 3
        idx2 = (scratch[o:o + 2 * NG], scratch[o + 2 * NG:o + 4 * NG])
        o += 4 * NG
        w2 = (scratch[o:o + 4], scratch[o + 4:o + 8])
        o += 8
        r2 = (scratch[o:o + 2 * NG], scratch[o + 2 * NG:o + 4 * NG])
        o += 4 * NG
        out2 = scratch[o:o + 2]
        gsem = scratch[o + 2:o + 4]
        osem = scratch[o + 4:o + 6]
        _sc_body(px_hbm, py_hbm, pz_hbm, table_hbm, out_hbm,
                 cx, cy, cz, idx2, w2, r2, out2, gsem, osem,
                 pts_per_w=pts_per_w, base_step=base_step, base_max=base_max)

    kern = pl.kernel(
        full_body,
        out_type=jax.ShapeDtypeStruct((n, OC), jnp.float32),
        mesh=mesh,
        compiler_params=pltpu.CompilerParams(use_tc_tiling_on_sc=True),
        scratch_types=(
            [pltpu.VMEM((pts_per_w,), jnp.float32)] * 3
            + [pltpu.VMEM((L,), jnp.int32)] * (4 * NG)
            + [pltpu.VMEM((PB,), jnp.float32)] * 8
            + [pltpu.VMEM((L, C), jnp.float32)] * (4 * NG)
            + [pltpu.VMEM((PB, OC), jnp.float32)] * 2
            + [pltpu.SemaphoreType.DMA] * 4
        ),
    )
    return kern(px, py, pz, table)


def kernel(p, c_xz, c_xy, c_yz):
    n = p.shape[1]
    pts_per_w = 2 * PB * max(1, (n + 2 * PB * NW - 1) // (2 * PB * NW))

    tabs = [jnp.transpose(cc[0].astype(jnp.bfloat16),
                          (1, 2, 0)).reshape(H * W, C)
            for cc in (c_xz, c_xy, c_yz)]
    tabs = [jnp.concatenate([t, jnp.roll(t, -1, axis=0)], axis=1)
            for t in tabs]
    table = jax.lax.bitcast_convert_type(
        jnp.concatenate(tabs, axis=0).reshape(PLANES * H * W, C, 2),
        jnp.float32)

    px, py, pz = p[0, :, 0], p[0, :, 1], p[0, :, 2]
    out = _sc_sample(px, py, pz, table, n, pts_per_w)
    out = jax.lax.bitcast_convert_type(out, jnp.bfloat16)
    return out.reshape(n, PLANES * C).astype(jnp.float32)[None]


# speedup vs baseline: 1.3708x; 1.3708x over previous
import functools

import jax
import jax.numpy as jnp
import numpy as np
from jax import lax
from jax.experimental import pallas as pl
from jax.experimental.pallas import tpu as pltpu
from jax.experimental.pallas import tpu_sc as plsc

NC = 2
NS = 16
L = 16
NW = NC * NS

C = 128
H = 128
W = 128
PB = 64
NG = PB // L
PLANES = 3
OB = PB * PLANES * C

_DEN = 1 + 0.1 + 10e-4
_CLAMP_HI = 1 - 10e-4

OC = PLANES * C // 2

_PERM = np.arange(C).reshape(2, C // 2).T.ravel()


def _coords_to_cells(a):
    u = a / jnp.float32(_DEN)
    u = u + jnp.float32(0.5)
    u = jnp.where(u >= jnp.float32(1.0), jnp.float32(_CLAMP_HI), u)
    u = jnp.where(u < jnp.float32(0.0), jnp.float32(0.0), u)
    vg = jnp.float32(2.0) * u - jnp.float32(1.0)
    x = (vg + jnp.float32(1.0)) * jnp.float32(0.5) * jnp.float32(W - 1)
    x = jnp.minimum(jnp.maximum(x, jnp.float32(0.0)), jnp.float32(W - 1))
    c0 = x.astype(jnp.int32)
    w = x - c0.astype(jnp.float32)
    c0 = jnp.minimum(c0, jnp.int32(W - 2))
    return c0, w


def _bf16x2(v):
    u = lax.bitcast_convert_type(v, jnp.int32)
    lo = lax.bitcast_convert_type(u << jnp.int32(16), jnp.float32)
    hi = lax.bitcast_convert_type(u & jnp.int32(-65536), jnp.float32)
    return lo, hi


def _sc_body(px_hbm, py_hbm, pz_hbm, table_hbm, out_hbm,
             cx, cy, cz, idx2, w2, r2, out2, gsem, osem,
             *, pts_per_w, base_step, base_max):
    wid = lax.axis_index("s") * NC + lax.axis_index("c")
    base = pl.multiple_of(
        jnp.minimum(wid * base_step, base_max) & jnp.int32(-8), 8)
    nblk = pts_per_w // PB
    npair = nblk // 2

    pltpu.sync_copy(px_hbm.at[pl.ds(base, pts_per_w)], cx)
    pltpu.sync_copy(py_hbm.at[pl.ds(base, pts_per_w)], cy)
    pltpu.sync_copy(pz_hbm.at[pl.ds(base, pts_per_w)], cz)

    plane_coords = [(cx, cz), (cx, cy), (cy, cz)]

    def compute_idx(par, plane, loc):
        ca, cb = plane_coords[plane]
        w00, w01, w10, w11 = w2[par]
        for g in range(NG):
            sl = pl.ds(loc + g * L, L)
            so = pl.ds(g * L, L)
            sf = pl.ds(0, L)
            x0, wx = _coords_to_cells(ca[sl])
            y0, wy = _coords_to_cells(cb[sl])
            idx = (y0 * W + x0) + jnp.int32(plane * H * W)
            idx2[par][g][sf] = idx
            idx2[par][NG + g][sf] = idx + W
            one = jnp.float32(1.0)
            w00[so] = (one - wx) * (one - wy)
            w01[so] = wx * (one - wy)
            w10[so] = (one - wx) * wy
            w11[so] = wx * wy

    def fire(par):
        for k in range(2 * NG):
            pltpu.make_async_copy(table_hbm.at[idx2[par][k]], r2[par][k],
                                  gsem[par]).start()

    def drain(par):
        for k in range(2 * NG):
            pltpu.make_async_copy(table_hbm.at[idx2[par][k]], r2[par][k],
                                  gsem[par]).wait()

    def consume(par, plane, out_v):
        w00, w01, w10, w11 = w2[par]
        for g in range(NG):
            sg = pl.ds(g * L, L)
            wv00 = w00[sg]
            wv01 = w01[sg]
            wv10 = w10[sg]
            wv11 = w11[sg]
            r0 = r2[par][g]
            r1 = r2[par][NG + g]

            def pt_body(ii, _, plane=plane, g=g, r0=r0, r1=r1,
                        wv00=wv00, wv01=wv01, wv10=wv10, wv11=wv11):
                i16 = jnp.full((L,), ii, jnp.int32)
                w00s = wv00.at[i16].get(mode="promise_in_bounds")
                w01s = wv01.at[i16].get(mode="promise_in_bounds")
                w10s = wv10.at[i16].get(mode="promise_in_bounds")
                w11s = wv11.at[i16].get(mode="promise_in_bounds")
                i = g * L + ii
                for j in range(C // (2 * L)):
                    cs = pl.ds(j * L, L)
                    cs1 = pl.ds(C // 2 + j * L, L)
                    a00, b00 = _bf16x2(r0[ii, cs])
                    a01, b01 = _bf16x2(r0[ii, cs1])
                    a10, b10 = _bf16x2(r1[ii, cs])
                    a11, b11 = _bf16x2(r1[ii, cs1])
                    acc_a = (w00s * a00 + w01s * a01
                             + w10s * a10 + w11s * a11)
                    acc_b = (w00s * b00 + w01s * b01
                             + w10s * b10 + w11s * b11)
                    ua = lax.bitcast_convert_type(acc_a, jnp.int32)
                    ub = lax.bitcast_convert_type(acc_b, jnp.int32)
                    half = jnp.int32(0x8000)
                    lo16 = ((ua + half) >> jnp.int32(16)) & jnp.int32(0xFFFF)
                    hi16 = (ub + half) & jnp.int32(-65536)
                    pk = lax.bitcast_convert_type(lo16 | hi16, jnp.float32)
                    out_v[i, pl.ds(plane * (C // 2) + j * L, L)] = pk
                return 0

            lax.fori_loop(0, L, pt_body, 0)

    compute_idx(0, 0, 0)
    fire(0)

    def pair_body(i, _):
        for k in range(6):
            plane = k % 3
            boff = k // 3
            par = k % 2
            nxt = (k + 1) % 2
            blk = 2 * i + boff
            out_v = out2[boff]
            nplane = (k + 1) % 3
            nboff = (k + 1) // 3
            if k < 5:
                compute_idx(nxt, nplane, (2 * i + nboff) * PB)
                fire(nxt)
            else:
                @pl.when(i < npair - 1)
                def _():
                    compute_idx(nxt, 0, (2 * i + 2) * PB)
                    fire(nxt)
            drain(par)
            if plane == 0:
                @pl.when(i > 0)
                def _():
                    pltpu.make_async_copy(
                        out_v, out_hbm.at[pl.ds(base, PB)], osem[boff]).wait()
            consume(par, plane, out_v)
            if plane == 2:
                pltpu.make_async_copy(
                    out_v, out_hbm.at[pl.ds(base + blk * PB, PB)],
                    osem[boff]).start()
        return 0

    lax.fori_loop(0, npair, pair_body, 0)

    for boff in range(2):
        pltpu.make_async_copy(out2[boff], out_hbm.at[pl.ds(base, PB)],
                              osem[boff]).wait()


@functools.partial(jax.jit, static_argnames=("n", "pts_per_w"))
def _sc_sample(px, py, pz, table, n, pts_per_w):
    base_step = -(-(n - pts_per_w) // (NW - 1))
    base_max = n - pts_per_w
    mesh = plsc.VectorSubcoreMesh(
        core_axis_name="c", subcore_axis_name="s", num_cores=NC,
        num_subcores=NS)

    def full_body(px_hbm, py_hbm, pz_hbm, table_hbm, out_hbm, *scratch):
        cx, cy, cz = scratch[0:3]
        o = 3
        idx2 = (scratch[o:o + 2 * NG], scratch[o + 2 * NG:o + 4 * NG])
        o += 4 * NG
        w2 = (scratch[o:o + 4], scratch[o + 4:o + 8])
        o += 8
        r2 = (scratch[o:o + 2 * NG], scratch[o + 2 * NG:o + 4 * NG])
        o += 4 * NG
        out2 = scratch[o:o + 2]
        gsem = scratch[o + 2:o + 4]
        osem = scratch[o + 4:o + 6]
        _sc_body(px_hbm, py_hbm, pz_hbm, table_hbm, out_hbm,
                 cx, cy, cz, idx2, w2, r2, out2, gsem, osem,
                 pts_per_w=pts_per_w, base_step=base_step, base_max=base_max)

    kern = pl.kernel(
        full_body,
        out_type=jax.ShapeDtypeStruct((n, OC), jnp.float32),
        mesh=mesh,
        compiler_params=pltpu.CompilerParams(use_tc_tiling_on_sc=True),
        scratch_types=(
            [pltpu.VMEM((pts_per_w,), jnp.float32)] * 3
            + [pltpu.VMEM((L,), jnp.int32)] * (4 * NG)
            + [pltpu.VMEM((PB,), jnp.float32)] * 8
            + [pltpu.VMEM((L, C), jnp.float32)] * (4 * NG)
            + [pltpu.VMEM((PB, OC), jnp.float32)] * 2
            + [pltpu.SemaphoreType.DMA] * 4
        ),
    )
    return kern(px, py, pz, table)


def kernel(p, c_xz, c_xy, c_yz):
    n = p.shape[1]
    pts_per_w = 2 * PB * max(1, (n + 2 * PB * NW - 1) // (2 * PB * NW))

    tabs = [jnp.transpose(cc[0][_PERM].astype(jnp.bfloat16),
                          (1, 2, 0)).reshape(H * W, C)
            for cc in (c_xz, c_xy, c_yz)]
    tabs = [jnp.concatenate([t, jnp.roll(t, -1, axis=0)], axis=1)
            for t in tabs]
    table = jax.lax.bitcast_convert_type(
        jnp.concatenate(tabs, axis=0).reshape(PLANES * H * W, C, 2),
        jnp.float32)

    px, py, pz = p[0, :, 0], p[0, :, 1], p[0, :, 2]
    out = _sc_sample(px, py, pz, table, n, pts_per_w)
    u = jax.lax.bitcast_convert_type(out, jnp.int32)
    lo = jax.lax.bitcast_convert_type(u << 16, jnp.float32)
    hi = jax.lax.bitcast_convert_type(u & jnp.int32(-65536), jnp.float32)
    hc = C // 2
    parts = []
    for q in range(PLANES):
        parts.append(lo[:, q * hc:(q + 1) * hc])
        parts.append(hi[:, q * hc:(q + 1) * hc])
    return jnp.concatenate(parts, axis=1)[None]


# speedup vs baseline: 2.5687x; 1.8738x over previous
import functools

import jax
import jax.numpy as jnp
import numpy as np
from jax import lax
from jax.experimental import pallas as pl
from jax.experimental.pallas import tpu as pltpu
from jax.experimental.pallas import tpu_sc as plsc

NC = 2
NS = 16
L = 16
NW = NC * NS

C = 128
H = 128
W = 128
PB = 64
NG = PB // L
PLANES = 3
OB = PB * PLANES * C

_DEN = 1 + 0.1 + 10e-4
_CLAMP_HI = 1 - 10e-4

_PERM = np.concatenate(
    [g * 32 + np.arange(32).reshape(2, 16).T.ravel() for g in range(4)])


def _coords_to_cells(a):
    u = a / jnp.float32(_DEN)
    u = u + jnp.float32(0.5)
    u = jnp.where(u >= jnp.float32(1.0), jnp.float32(_CLAMP_HI), u)
    u = jnp.where(u < jnp.float32(0.0), jnp.float32(0.0), u)
    vg = jnp.float32(2.0) * u - jnp.float32(1.0)
    x = (vg + jnp.float32(1.0)) * jnp.float32(0.5) * jnp.float32(W - 1)
    x = jnp.minimum(jnp.maximum(x, jnp.float32(0.0)), jnp.float32(W - 1))
    c0 = x.astype(jnp.int32)
    w = x - c0.astype(jnp.float32)
    c0 = jnp.minimum(c0, jnp.int32(W - 2))
    return c0, w


def _bf16x2(v):
    u = lax.bitcast_convert_type(v, jnp.int32)
    lo = lax.bitcast_convert_type(u << jnp.int32(16), jnp.float32)
    hi = lax.bitcast_convert_type(u & jnp.int32(-65536), jnp.float32)
    return lo, hi


def _sc_body(px_hbm, py_hbm, pz_hbm, table_hbm, out_hbm,
             cx, cy, cz, idx2, w2, r2, out2, gsem, osem,
             *, pts_per_w, base_step, base_max):
    wid = lax.axis_index("s") * NC + lax.axis_index("c")
    base = pl.multiple_of(
        jnp.minimum(wid * base_step, base_max) & jnp.int32(-8), 8)
    nblk = pts_per_w // PB
    npair = nblk // 2

    pltpu.sync_copy(px_hbm.at[pl.ds(base, pts_per_w)], cx)
    pltpu.sync_copy(py_hbm.at[pl.ds(base, pts_per_w)], cy)
    pltpu.sync_copy(pz_hbm.at[pl.ds(base, pts_per_w)], cz)

    plane_coords = [(cx, cz), (cx, cy), (cy, cz)]

    def compute_idx(par, plane, loc):
        ca, cb = plane_coords[plane]
        w00, w01, w10, w11 = w2[par]
        for g in range(NG):
            sl = pl.ds(loc + g * L, L)
            so = pl.ds(g * L, L)
            sf = pl.ds(0, L)
            x0, wx = _coords_to_cells(ca[sl])
            y0, wy = _coords_to_cells(cb[sl])
            idx = (y0 * W + x0) + jnp.int32(plane * H * W)
            idx2[par][g][sf] = idx
            idx2[par][NG + g][sf] = idx + W
            one = jnp.float32(1.0)
            w00[so] = (one - wx) * (one - wy)
            w01[so] = wx * (one - wy)
            w10[so] = (one - wx) * wy
            w11[so] = wx * wy

    def fire(par):
        for k in range(2 * NG):
            pltpu.make_async_copy(table_hbm.at[idx2[par][k]], r2[par][k],
                                  gsem[par]).start()

    def drain(par):
        for k in range(2 * NG):
            pltpu.make_async_copy(table_hbm.at[idx2[par][k]], r2[par][k],
                                  gsem[par]).wait()

    def consume(par, plane, out_v):
        w00, w01, w10, w11 = w2[par]
        for g in range(NG):
            sg = pl.ds(g * L, L)
            wv00 = w00[sg]
            wv01 = w01[sg]
            wv10 = w10[sg]
            wv11 = w11[sg]
            r0 = r2[par][g]
            r1 = r2[par][NG + g]

            @plsc.parallel_loop(0, L, 1, unroll=2)
            def pt_body(ii, plane=plane, g=g, r0=r0, r1=r1,
                        wv00=wv00, wv01=wv01, wv10=wv10, wv11=wv11):
                i16 = jnp.full((L,), ii, jnp.int32)
                w00s = wv00.at[i16].get(mode="promise_in_bounds")
                w01s = wv01.at[i16].get(mode="promise_in_bounds")
                w10s = wv10.at[i16].get(mode="promise_in_bounds")
                w11s = wv11.at[i16].get(mode="promise_in_bounds")
                i = g * L + ii
                for j in range(C // (2 * L)):
                    cs = pl.ds(j * L, L)
                    cs1 = pl.ds(C // 2 + j * L, L)
                    a00, b00 = _bf16x2(r0[ii, cs])
                    a01, b01 = _bf16x2(r0[ii, cs1])
                    a10, b10 = _bf16x2(r1[ii, cs])
                    a11, b11 = _bf16x2(r1[ii, cs1])
                    acc_a = (w00s * a00 + w01s * a01
                             + w10s * a10 + w11s * a11)
                    acc_b = (w00s * b00 + w01s * b01
                             + w10s * b10 + w11s * b11)
                    out_v[i, pl.ds(plane * C + j * 2 * L, L)] = acc_a
                    out_v[i, pl.ds(plane * C + j * 2 * L + L, L)] = acc_b

    compute_idx(0, 0, 0)
    fire(0)

    def pair_body(i, _):
        for k in range(6):
            plane = k % 3
            boff = k // 3
            par = k % 2
            nxt = (k + 1) % 2
            blk = 2 * i + boff
            out_v = out2[boff]
            nplane = (k + 1) % 3
            nboff = (k + 1) // 3
            if k < 5:
                compute_idx(nxt, nplane, (2 * i + nboff) * PB)
                fire(nxt)
            else:
                @pl.when(i < npair - 1)
                def _():
                    compute_idx(nxt, 0, (2 * i + 2) * PB)
                    fire(nxt)
            drain(par)
            if plane == 0:
                @pl.when(i > 0)
                def _():
                    pltpu.make_async_copy(
                        out_v, out_hbm.at[pl.ds(base, PB)], osem[boff]).wait()
            consume(par, plane, out_v)
            if plane == 2:
                pltpu.make_async_copy(
                    out_v, out_hbm.at[pl.ds(base + blk * PB, PB)],
                    osem[boff]).start()
        return 0

    lax.fori_loop(0, npair, pair_body, 0)

    for boff in range(2):
        pltpu.make_async_copy(out2[boff], out_hbm.at[pl.ds(base, PB)],
                              osem[boff]).wait()


@functools.partial(jax.jit, static_argnames=("n", "pts_per_w"))
def _sc_sample(px, py, pz, table, n, pts_per_w):
    base_step = -(-(n - pts_per_w) // (NW - 1))
    base_max = n - pts_per_w
    mesh = plsc.VectorSubcoreMesh(
        core_axis_name="c", subcore_axis_name="s", num_cores=NC,
        num_subcores=NS)

    def full_body(px_hbm, py_hbm, pz_hbm, table_hbm, out_hbm, *scratch):
        cx, cy, cz = scratch[0:3]
        o = 3
        idx2 = (scratch[o:o + 2 * NG], scratch[o + 2 * NG:o + 4 * NG])
        o += 4 * NG
        w2 = (scratch[o:o + 4], scratch[o + 4:o + 8])
        o += 8
        r2 = (scratch[o:o + 2 * NG], scratch[o + 2 * NG:o + 4 * NG])
        o += 4 * NG
        out2 = scratch[o:o + 2]
        gsem = scratch[o + 2:o + 4]
        osem = scratch[o + 4:o + 6]
        _sc_body(px_hbm, py_hbm, pz_hbm, table_hbm, out_hbm,
                 cx, cy, cz, idx2, w2, r2, out2, gsem, osem,
                 pts_per_w=pts_per_w, base_step=base_step, base_max=base_max)

    kern = pl.kernel(
        full_body,
        out_type=jax.ShapeDtypeStruct((n, PLANES * C), jnp.float32),
        mesh=mesh,
        compiler_params=pltpu.CompilerParams(use_tc_tiling_on_sc=True),
        scratch_types=(
            [pltpu.VMEM((pts_per_w,), jnp.float32)] * 3
            + [pltpu.VMEM((L,), jnp.int32)] * (4 * NG)
            + [pltpu.VMEM((PB,), jnp.float32)] * 8
            + [pltpu.VMEM((L, C), jnp.float32)] * (4 * NG)
            + [pltpu.VMEM((PB, PLANES * C), jnp.float32)] * 2
            + [pltpu.SemaphoreType.DMA] * 4
        ),
    )
    return kern(px, py, pz, table)


def kernel(p, c_xz, c_xy, c_yz):
    n = p.shape[1]
    pts_per_w = 2 * PB * max(1, (n + 2 * PB * NW - 1) // (2 * PB * NW))

    tabs = [jnp.transpose(cc[0][_PERM].astype(jnp.bfloat16),
                          (1, 2, 0)).reshape(H * W, C)
            for cc in (c_xz, c_xy, c_yz)]
    tabs = [jnp.concatenate([t, jnp.roll(t, -1, axis=0)], axis=1)
            for t in tabs]
    table = jax.lax.bitcast_convert_type(
        jnp.concatenate(tabs, axis=0).reshape(PLANES * H * W, C, 2),
        jnp.float32)

    px, py, pz = p[0, :, 0], p[0, :, 1], p[0, :, 2]
    out = _sc_sample(px, py, pz, table, n, pts_per_w)
    return out[None]
